# baseline (speedup 1.0000x reference)
import functools

import jax
import jax.numpy as jnp
from jax import lax
from jax.experimental import pallas as pl
from jax.experimental.pallas import tpu as pltpu
from jax.experimental.pallas import tpu_sc as plsc

BATCH = 16384
HID = 64
NC = 2
NS = 16
NW = NC * NS
RPW = BATCH // NW
CHUNK = 128
NCH = RPW // CHUNK
GROUPS = RPW // 16


def _sc_body(user_hbm, item_hbm, uw_hbm, iw_hbm, ub_hbm, ib_hbm,
             out_hbm, uidx, iidx, urows, irows, ubv, ibv, outv, sem):
    wid = lax.axis_index("s") * NC + lax.axis_index("c")
    base = wid * RPW

    pltpu.sync_copy(user_hbm.at[pl.ds(base, RPW)], uidx)
    pltpu.sync_copy(item_hbm.at[pl.ds(base, RPW)], iidx)

    copies = []
    for k in range(NCH):
        sl = pl.ds(k * CHUNK, CHUNK)
        copies.append(pltpu.async_copy(uw_hbm.at[uidx.at[sl]], urows.at[sl], sem))
        copies.append(pltpu.async_copy(iw_hbm.at[iidx.at[sl]], irows.at[sl], sem))
        copies.append(pltpu.async_copy(ub_hbm.at[uidx.at[sl]], ubv.at[sl], sem))
        copies.append(pltpu.async_copy(ib_hbm.at[iidx.at[sl]], ibv.at[sl], sem))
    for c in copies:
        c.wait()

    def group(g, carry):
        gb = g * 16
        ub_g = ubv[pl.ds(gb, 16)]
        ib_g = ibv[pl.ds(gb, 16)]
        for j in range(16):
            r = gb + j
            ubs = ub_g[j]
            ibs = ib_g[j]
            acc = None
            for c in range(HID // 16):
                u = urows[r, pl.ds(c * 16, 16)] + ubs
                v = irows[r, pl.ds(c * 16, 16)] + ibs
                p = u * v
                acc = p if acc is None else acc + p
            outv[r, pl.ds(0, 16)] = acc
        return carry

    lax.fori_loop(0, GROUPS, group, None)

    pltpu.sync_copy(outv, out_hbm.at[pl.ds(base, RPW)])


@jax.jit
def _mf(user, item, uw, iw, ub_flat, ib_flat, bias):
    mesh = plsc.VectorSubcoreMesh(core_axis_name="c", subcore_axis_name="s")
    sc_run = pl.kernel(
        _sc_body,
        out_type=jax.ShapeDtypeStruct((BATCH, 16), jnp.float32),
        mesh=mesh,
        scratch_types=[
            pltpu.VMEM((RPW,), jnp.int32),
            pltpu.VMEM((RPW,), jnp.int32),
            pltpu.VMEM((RPW, HID), jnp.float32),
            pltpu.VMEM((RPW, HID), jnp.float32),
            pltpu.VMEM((RPW,), jnp.float32),
            pltpu.VMEM((RPW,), jnp.float32),
            pltpu.VMEM((RPW, 16), jnp.float32),
            pltpu.SemaphoreType.DMA,
        ],
        compiler_params=pltpu.CompilerParams(use_tc_tiling_on_sc=False),
    )
    pacc = sc_run(user, item, uw, iw, ub_flat, ib_flat)

    p2 = pacc.reshape(BATCH // 8, 128)
    bias_row = bias.reshape(1, 1)

    def tc_body(p_ref, b_ref, o_ref):
        l_ids = lax.broadcasted_iota(jnp.int32, (128, 8), 0)
        g_ids = lax.broadcasted_iota(jnp.int32, (128, 8), 1)
        m = jnp.where(l_ids // 16 == g_ids, 1.0, 0.0).astype(jnp.float32)
        red = jnp.dot(p_ref[...], m, preferred_element_type=jnp.float32)
        o_ref[...] = red + b_ref[0, 0]

    out2 = pl.pallas_call(
        tc_body,
        out_shape=jax.ShapeDtypeStruct((BATCH // 8, 8), jnp.float32),
        in_specs=[
            pl.BlockSpec((BATCH // 8, 128), lambda: (0, 0)),
            pl.BlockSpec(memory_space=pltpu.SMEM),
        ],
        out_specs=pl.BlockSpec((BATCH // 8, 8), lambda: (0, 0)),
    )(p2, bias_row)
    return out2.reshape(BATCH)


def kernel(user, item, target, user_weight, item_weight, user_bias, item_bias,
           bias):
    del target
    ub_flat = user_bias.reshape(-1)
    ib_flat = item_bias.reshape(-1)
    return _mf(user, item, user_weight, item_weight, ub_flat, ib_flat, bias)
